# baseline (speedup 1.0000x reference)
import functools

import jax
import jax.numpy as jnp
from jax import lax
from jax.experimental import pallas as pl
from jax.experimental.pallas import tpu as pltpu
from jax.experimental.pallas import tpu_sc as plsc

EMB = 32
BATCH = 16384

_NC = 2
_NS = 16
_NW = _NC * _NS
_BPW = BATCH // _NW
_CHUNK = 128
_NCHUNK = _BPW // _CHUNK


def _gather_body(table_hbm, idx_hbm, out_hbm, idx_v, rows_v, sem):
    wid = lax.axis_index("s") * _NC + lax.axis_index("c")
    pltpu.sync_copy(idx_hbm.at[pl.ds(wid * _NCHUNK, _NCHUNK)], idx_v)
    copies = [
        pltpu.async_copy(
            table_hbm.at[idx_v.at[j]],
            rows_v.at[pl.ds(j * _CHUNK, _CHUNK)],
            sem,
        )
        for j in range(_NCHUNK)
    ]
    for c in copies:
        c.wait()
    pltpu.sync_copy(rows_v, out_hbm.at[pl.ds(wid * _BPW, _BPW)])


def _sc_gather(table, idx2d):
    mesh = plsc.VectorSubcoreMesh(core_axis_name="c", subcore_axis_name="s")
    fn = functools.partial(
        pl.kernel,
        mesh=mesh,
        out_type=jax.ShapeDtypeStruct((BATCH, EMB), jnp.float32),
        scratch_types=[
            pltpu.VMEM((_NCHUNK, _CHUNK), jnp.int32),
            pltpu.VMEM((_BPW, EMB), jnp.float32),
            pltpu.SemaphoreType.DMA,
        ],
        compiler_params=pltpu.CompilerParams(use_tc_tiling_on_sc=False),
    )(_gather_body)
    return fn(table, idx2d)


_BB = 2048


def _mlp_body(x_ref, w1_ref, b1_ref, w2_ref, b2_ref, w3_ref, b3_ref, o_ref):
    x = x_ref[...]
    h = jnp.dot(x, w1_ref[...], preferred_element_type=jnp.float32) + b1_ref[...]
    h = jnp.maximum(h, 0.0)
    h = jnp.dot(h, w2_ref[...], preferred_element_type=jnp.float32) + b2_ref[...]
    h = jnp.maximum(h, 0.0)
    o_ref[...] = (
        jnp.dot(h, w3_ref[...], preferred_element_type=jnp.float32) + b3_ref[...]
    )


def _mlp(x, W1, b1, W2, b2, W3, b3):
    grid = (BATCH // _BB,)
    full = lambda shape: pl.BlockSpec(shape, lambda i: (0, 0))
    return pl.pallas_call(
        _mlp_body,
        grid=grid,
        in_specs=[
            pl.BlockSpec((_BB, EMB), lambda i: (i, 0)),
            full(W1.shape),
            full(b1.shape),
            full(W2.shape),
            full(b2.shape),
            full(W3.shape),
            full(b3.shape),
        ],
        out_specs=pl.BlockSpec((_BB, 64), lambda i: (i, 0)),
        out_shape=jax.ShapeDtypeStruct((BATCH, 64), jnp.float32),
    )(x, W1, b1, W2, b2, W3, b3)


def kernel(item_idx, table, W1, b1, W2, b2, W3, b3):
    idx2d = item_idx.astype(jnp.int32).reshape(_NW * _NCHUNK, _CHUNK)
    x = _sc_gather(table, idx2d)
    return _mlp(
        x,
        W1,
        b1.reshape(1, -1),
        W2,
        b2.reshape(1, -1),
        W3,
        b3.reshape(1, -1),
    )


# speedup vs baseline: 1.6396x; 1.6396x over previous
import functools

import jax
import jax.numpy as jnp
from jax import lax
from jax.experimental import pallas as pl
from jax.experimental.pallas import tpu as pltpu
from jax.experimental.pallas import tpu_sc as plsc

EMB = 32
BATCH = 16384

_NC = 2
_NS = 16
_NW = _NC * _NS
_BPW = BATCH // _NW
_CHUNK = 128
_NCHUNK = _BPW // _CHUNK


_UNROLL = 16


def _gather_body(table_hbm, idx_hbm, out_hbm, idx_v, rows_v, sem):
    wid = lax.axis_index("s") * _NC + lax.axis_index("c")
    base = wid * _BPW
    pltpu.sync_copy(idx_hbm.at[pl.ds(base, _BPW)], idx_v)

    def fire(i, carry):
        v = idx_v[pl.ds(i * _UNROLL, _UNROLL)]
        for k in range(_UNROLL):
            pltpu.async_copy(
                table_hbm.at[pl.ds(v[k], 1)],
                rows_v.at[pl.ds(i * _UNROLL + k, 1)],
                sem,
            )
        return carry

    lax.fori_loop(0, _BPW // _UNROLL, fire, 0, unroll=False)
    pltpu.make_async_copy(table_hbm.at[pl.ds(0, _BPW)], rows_v, sem).wait()
    pltpu.sync_copy(rows_v, out_hbm.at[pl.ds(base, _BPW)])


def _sc_gather(table, idx):
    mesh = plsc.VectorSubcoreMesh(core_axis_name="c", subcore_axis_name="s")
    fn = functools.partial(
        pl.kernel,
        mesh=mesh,
        out_type=jax.ShapeDtypeStruct((BATCH, EMB), jnp.float32),
        scratch_types=[
            pltpu.VMEM((_BPW,), jnp.int32),
            pltpu.VMEM((_BPW, EMB), jnp.float32),
            pltpu.SemaphoreType.DMA,
        ],
    )(_gather_body)
    return fn(table, idx)


_BB = 2048


def _mlp_body(x_ref, w1_ref, b1_ref, w2_ref, b2_ref, w3_ref, b3_ref, o_ref):
    x = x_ref[...]
    h = jnp.dot(x, w1_ref[...], preferred_element_type=jnp.float32) + b1_ref[...]
    h = jnp.maximum(h, 0.0)
    h = jnp.dot(h, w2_ref[...], preferred_element_type=jnp.float32) + b2_ref[...]
    h = jnp.maximum(h, 0.0)
    o_ref[...] = (
        jnp.dot(h, w3_ref[...], preferred_element_type=jnp.float32) + b3_ref[...]
    )


def _mlp(x, W1, b1, W2, b2, W3, b3):
    grid = (BATCH // _BB,)
    full = lambda shape: pl.BlockSpec(shape, lambda i: (0, 0))
    return pl.pallas_call(
        _mlp_body,
        grid=grid,
        in_specs=[
            pl.BlockSpec((_BB, EMB), lambda i: (i, 0)),
            full(W1.shape),
            full(b1.shape),
            full(W2.shape),
            full(b2.shape),
            full(W3.shape),
            full(b3.shape),
        ],
        out_specs=pl.BlockSpec((_BB, 64), lambda i: (i, 0)),
        out_shape=jax.ShapeDtypeStruct((BATCH, 64), jnp.float32),
    )(x, W1, b1, W2, b2, W3, b3)


def kernel(item_idx, table, W1, b1, W2, b2, W3, b3):
    x = _sc_gather(table, item_idx.astype(jnp.int32))
    return _mlp(
        x,
        W1,
        b1.reshape(1, -1),
        W2,
        b2.reshape(1, -1),
        W3,
        b3.reshape(1, -1),
    )


# speedup vs baseline: 1.9369x; 1.1813x over previous
import functools

import jax
import jax.numpy as jnp
from jax import lax
from jax.experimental import pallas as pl
from jax.experimental.pallas import tpu as pltpu
from jax.experimental.pallas import tpu_sc as plsc

NITEMS = 1000000
EMB = 32
BATCH = 16384


_LB = 8192


def _tr_body(tT_ref, o_ref):
    blk = tT_ref[...]
    eye = jnp.float32(
        lax.broadcasted_iota(jnp.int32, (EMB, EMB), 0)
        == lax.broadcasted_iota(jnp.int32, (EMB, EMB), 1)
    )
    o_ref[...] = lax.dot_general(
        blk, eye, (((0,), (0,)), ((), ())), preferred_element_type=jnp.float32
    )


def _relayout(tableT):
    grid = (pl.cdiv(NITEMS, _LB),)
    return pl.pallas_call(
        _tr_body,
        grid=grid,
        in_specs=[pl.BlockSpec((EMB, _LB), lambda i: (0, i))],
        out_specs=pl.BlockSpec((_LB, EMB), lambda i: (i, 0)),
        out_shape=jax.ShapeDtypeStruct((NITEMS, EMB), jnp.float32),
    )(tableT)


_NC = 2
_NS = 16
_NW = _NC * _NS
_BPW = BATCH // _NW
_UNROLL = 16


def _gather_body(table_hbm, idx_hbm, out_hbm, idx_v, rows_v, sem):
    wid = lax.axis_index("s") * _NC + lax.axis_index("c")
    base = wid * _BPW
    pltpu.sync_copy(idx_hbm.at[pl.ds(base, _BPW)], idx_v)

    def fire(i, carry):
        v = idx_v[pl.ds(i * _UNROLL, _UNROLL)]
        for k in range(_UNROLL):
            pltpu.async_copy(
                table_hbm.at[pl.ds(v[k], 1)],
                rows_v.at[pl.ds(i * _UNROLL + k, 1)],
                sem,
            )
        return carry

    lax.fori_loop(0, _BPW // _UNROLL, fire, 0, unroll=False)
    pltpu.make_async_copy(table_hbm.at[pl.ds(0, _BPW)], rows_v, sem).wait()
    pltpu.sync_copy(rows_v, out_hbm.at[pl.ds(base, _BPW)])


def _sc_gather(table, idx):
    mesh = plsc.VectorSubcoreMesh(core_axis_name="c", subcore_axis_name="s")
    fn = functools.partial(
        pl.kernel,
        mesh=mesh,
        out_type=jax.ShapeDtypeStruct((BATCH, EMB), jnp.float32),
        scratch_types=[
            pltpu.VMEM((_BPW,), jnp.int32),
            pltpu.VMEM((_BPW, EMB), jnp.float32),
            pltpu.SemaphoreType.DMA,
        ],
    )(_gather_body)
    return fn(table, idx)


_BB = 2048


def _mlp_body(x_ref, w1_ref, b1_ref, w2_ref, b2_ref, w3_ref, b3_ref, oT_ref):
    x = x_ref[...]
    h = jnp.dot(x, w1_ref[...], preferred_element_type=jnp.float32) + b1_ref[...]
    h = jnp.maximum(h, 0.0)
    h = jnp.dot(h, w2_ref[...], preferred_element_type=jnp.float32) + b2_ref[...]
    h = jnp.maximum(h, 0.0)
    oT_ref[...] = (
        lax.dot_general(
            w3_ref[...], h, (((0,), (1,)), ((), ())),
            preferred_element_type=jnp.float32,
        )
        + b3_ref[...]
    )


def _mlp(x, W1, b1, W2, b2, W3, b3):
    grid = (BATCH // _BB,)
    full = lambda shape: pl.BlockSpec(shape, lambda i: (0, 0))
    return pl.pallas_call(
        _mlp_body,
        grid=grid,
        in_specs=[
            pl.BlockSpec((_BB, EMB), lambda i: (i, 0)),
            full(W1.shape),
            full(b1.shape),
            full(W2.shape),
            full(b2.shape),
            full(W3.shape),
            full(b3.shape),
        ],
        out_specs=pl.BlockSpec((64, _BB), lambda i: (0, i)),
        out_shape=jax.ShapeDtypeStruct((64, BATCH), jnp.float32),
    )(x, W1, b1, W2, b2, W3, b3)


def kernel(item_idx, table, W1, b1, W2, b2, W3, b3):
    table_rm = _relayout(table.T)
    x = _sc_gather(table_rm, item_idx.astype(jnp.int32))
    outT = _mlp(
        x,
        W1,
        b1.reshape(1, -1),
        W2,
        b2.reshape(1, -1),
        W3,
        b3.reshape(-1, 1),
    )
    return outT.T


# speedup vs baseline: 2.0360x; 1.0512x over previous
import functools

import jax
import jax.numpy as jnp
from jax import lax
from jax.experimental import pallas as pl
from jax.experimental.pallas import tpu as pltpu
from jax.experimental.pallas import tpu_sc as plsc

NITEMS = 1000000
EMB = 32
BATCH = 16384


_LB = 8192
_NBLK = -(-NITEMS // _LB)
_NPAD = _NBLK * _LB
_KQ = 8
_SEG = _LB // _KQ


def _tr_body(tT_ref, o_any, buf, sems):
    i = pl.program_id(0)
    slot = lax.rem(i, 2)

    def seg_copy(s, blk_i, k):
        return pltpu.make_async_copy(
            buf.at[s, pl.ds(k * _SEG, _SEG)],
            o_any.at[pl.ds(blk_i * _LB + k * _SEG, _SEG)],
            sems.at[s, k],
        )

    @pl.when(i >= 2)
    def _():
        for k in range(_KQ):
            seg_copy(slot, i - 2, k).wait()

    buf[slot] = tT_ref[...].T
    for k in range(_KQ):
        seg_copy(slot, i, k).start()

    @pl.when(i == _NBLK - 1)
    def _():
        s_last = (_NBLK - 1) % 2
        for k in range(_KQ):
            seg_copy(1 - s_last, _NBLK - 2, k).wait()
        for k in range(_KQ):
            seg_copy(s_last, _NBLK - 1, k).wait()


def _relayout(tableT):
    return pl.pallas_call(
        _tr_body,
        grid=(_NBLK,),
        in_specs=[pl.BlockSpec((EMB, _LB), lambda i: (0, i))],
        out_specs=pl.BlockSpec(memory_space=pl.ANY),
        out_shape=jax.ShapeDtypeStruct((_NPAD, EMB), jnp.float32),
        scratch_shapes=[
            pltpu.VMEM((2, _LB, EMB), jnp.float32),
            pltpu.SemaphoreType.DMA((2, _KQ)),
        ],
    )(tableT)


_NC = 2
_NS = 16
_NW = _NC * _NS
_BPW = BATCH // _NW
_UNROLL = 16


def _gather_body(table_hbm, idx_hbm, out_hbm, idx_v, rows_v, sem):
    wid = lax.axis_index("s") * _NC + lax.axis_index("c")
    base = wid * _BPW
    pltpu.sync_copy(idx_hbm.at[pl.ds(base, _BPW)], idx_v)

    def fire(i, carry):
        v = idx_v[pl.ds(i * _UNROLL, _UNROLL)]
        for k in range(_UNROLL):
            pltpu.async_copy(
                table_hbm.at[pl.ds(v[k], 1)],
                rows_v.at[pl.ds(i * _UNROLL + k, 1)],
                sem,
            )
        return carry

    lax.fori_loop(0, _BPW // _UNROLL, fire, 0, unroll=False)
    pltpu.make_async_copy(table_hbm.at[pl.ds(0, _BPW)], rows_v, sem).wait()
    pltpu.sync_copy(rows_v, out_hbm.at[pl.ds(base, _BPW)])


def _sc_gather(table, idx):
    mesh = plsc.VectorSubcoreMesh(core_axis_name="c", subcore_axis_name="s")
    fn = functools.partial(
        pl.kernel,
        mesh=mesh,
        out_type=jax.ShapeDtypeStruct((BATCH, EMB), jnp.float32),
        scratch_types=[
            pltpu.VMEM((_BPW,), jnp.int32),
            pltpu.VMEM((_BPW, EMB), jnp.float32),
            pltpu.SemaphoreType.DMA,
        ],
    )(_gather_body)
    return fn(table, idx)


_BB = 2048


def _mlp_body(x_ref, w1_ref, b1_ref, w2_ref, b2_ref, w3_ref, b3_ref, oT_ref):
    x = x_ref[...]
    h = jnp.dot(x, w1_ref[...], preferred_element_type=jnp.float32) + b1_ref[...]
    h = jnp.maximum(h, 0.0)
    h = jnp.dot(h, w2_ref[...], preferred_element_type=jnp.float32) + b2_ref[...]
    h = jnp.maximum(h, 0.0)
    oT_ref[...] = (
        lax.dot_general(
            w3_ref[...], h, (((0,), (1,)), ((), ())),
            preferred_element_type=jnp.float32,
        )
        + b3_ref[...]
    )


def _mlp(x, W1, b1, W2, b2, W3, b3):
    grid = (BATCH // _BB,)
    full = lambda shape: pl.BlockSpec(shape, lambda i: (0, 0))
    return pl.pallas_call(
        _mlp_body,
        grid=grid,
        in_specs=[
            pl.BlockSpec((_BB, EMB), lambda i: (i, 0)),
            full(W1.shape),
            full(b1.shape),
            full(W2.shape),
            full(b2.shape),
            full(W3.shape),
            full(b3.shape),
        ],
        out_specs=pl.BlockSpec((64, _BB), lambda i: (0, i)),
        out_shape=jax.ShapeDtypeStruct((64, BATCH), jnp.float32),
    )(x, W1, b1, W2, b2, W3, b3)


def kernel(item_idx, table, W1, b1, W2, b2, W3, b3):
    table_rm = _relayout(table.T)
    x = _sc_gather(table_rm, item_idx.astype(jnp.int32))
    outT = _mlp(
        x,
        W1,
        b1.reshape(1, -1),
        W2,
        b2.reshape(1, -1),
        W3,
        b3.reshape(-1, 1),
    )
    return outT.T
